# baseline (device time: 42279 ns/iter reference)
import jax
import jax.numpy as jnp
from jax import lax
from jax.experimental import pallas as pl
from jax.experimental.pallas import tpu as pltpu

N_DEV = 4
M = 256
D = 256


def kernel(x, Win0, Wout0, Win1, Wout1, Win2, Wout2):
    def body(x_ref, win0_ref, wout0_ref, win1_ref, wout1_ref, win2_ref,
             wout2_ref, out_ref, part_ref, red_ref, rbuf_ref,
             ag_stage_ref, rs_stage_ref, winb_ref, woutb_ref,
             send_sems, recv_sems):
        my = lax.axis_index("i")

        barrier_sem = pltpu.get_barrier_semaphore()
        for d in (1, 2, 3):
            pl.semaphore_signal(
                barrier_sem, inc=1,
                device_id=(lax.rem(my + d, N_DEV),),
                device_id_type=pl.DeviceIdType.MESH,
            )
        pl.semaphore_wait(barrier_sem, N_DEV - 1)

        def remote_copy(par, d, src):
            return pltpu.make_async_remote_copy(
                src_ref=src,
                dst_ref=rbuf_ref.at[par, N_DEV - d],
                send_sem=send_sems.at[par, d - 1],
                recv_sem=recv_sems.at[par, N_DEV - d],
                device_id=(lax.rem(my + d, N_DEV),),
                device_id_type=pl.DeviceIdType.MESH,
            )

        layers = ((win0_ref, wout0_ref), (win1_ref, wout1_ref),
                  (win2_ref, wout2_ref))
        for l, (win_ref, wout_ref) in enumerate(layers):
            src0_ref = x_ref if l == 0 else red_ref

            def block_compute(xblk_bf16, r, win=winb_ref, wout=woutb_ref):
                h = jnp.maximum(
                    jnp.dot(xblk_bf16, win[...],
                            preferred_element_type=jnp.float32),
                    0.0,
                )
                part_ref[r, :, :] = jnp.dot(
                    h.astype(jnp.bfloat16), wout[...],
                    preferred_element_type=jnp.float32)

            ag_stage_ref[...] = src0_ref[...].astype(jnp.bfloat16)
            ag = [remote_copy(0, d, ag_stage_ref) for d in (1, 2, 3)]
            for rdma in ag:
                rdma.start()

            winb_ref[...] = win_ref[...].astype(jnp.bfloat16)
            woutb_ref[...] = wout_ref[...].astype(jnp.bfloat16)

            block_compute(ag_stage_ref[...], 0)

            rs = []
            for r in (1, 3, 2):
                ag[3 - r].wait_recv()
                block_compute(rbuf_ref[0, r, :, :], r)
                rs_stage_ref[r - 1, :, :] = (
                    part_ref[r, :, :].astype(jnp.bfloat16))
                rdma = remote_copy(1, r, rs_stage_ref.at[r - 1])
                rdma.start()
                rs.append(rdma)

            for rdma in ag:
                rdma.wait_send()

            for rdma in rs:
                rdma.wait_recv()
            reduced = (
                part_ref[0, :, :]
                + rbuf_ref[1, 1, :, :].astype(jnp.float32)
                + rbuf_ref[1, 2, :, :].astype(jnp.float32)
                + rbuf_ref[1, 3, :, :].astype(jnp.float32)
            )
            if l < 2:
                red_ref[...] = reduced
            else:
                out_ref[...] = reduced

            for rdma in rs:
                rdma.wait_send()

    return pl.pallas_call(
        body,
        out_shape=jax.ShapeDtypeStruct((M, D), jnp.float32),
        in_specs=[pl.BlockSpec(memory_space=pltpu.VMEM)] * 7,
        out_specs=pl.BlockSpec(memory_space=pltpu.VMEM),
        scratch_shapes=[
            pltpu.VMEM((N_DEV, M, D), jnp.float32),
            pltpu.VMEM((M, D), jnp.float32),
            pltpu.VMEM((2, N_DEV, M, D), jnp.bfloat16),
            pltpu.VMEM((M, D), jnp.bfloat16),
            pltpu.VMEM((3, M, D), jnp.bfloat16),
            pltpu.VMEM((D, 2 * D), jnp.bfloat16),
            pltpu.VMEM((2 * D, D), jnp.bfloat16),
            pltpu.SemaphoreType.DMA((2, 3)),
            pltpu.SemaphoreType.DMA((2, N_DEV)),
        ],
        compiler_params=pltpu.CompilerParams(collective_id=0),
    )(x, Win0, Wout0, Win1, Wout1, Win2, Wout2)


# device time: 38437 ns/iter; 1.1000x vs baseline; 1.1000x over previous
import jax
import jax.numpy as jnp
from jax import lax
from jax.experimental import pallas as pl
from jax.experimental.pallas import tpu as pltpu

N_DEV = 4
M = 256
H = 128
D = 256


def kernel(x, Win0, Wout0, Win1, Wout1, Win2, Wout2):
    def body(x_ref, win0_ref, wout0_ref, win1_ref, wout1_ref, win2_ref,
             wout2_ref, out_ref, part_ref, rbuf_ref, ag_stage_ref,
             rs_stage_ref, winb_ref, woutb_ref,
             ag_ssems, rs_ssems, recv_sems):
        my = lax.axis_index("i")

        def rows(h):
            return pl.ds(h * H, H)

        barrier_sem = pltpu.get_barrier_semaphore()
        for d in (1, 2, 3):
            pl.semaphore_signal(
                barrier_sem, inc=1,
                device_id=(lax.rem(my + d, N_DEV),),
                device_id_type=pl.DeviceIdType.MESH,
            )
        pl.semaphore_wait(barrier_sem, N_DEV - 1)

        def ag_send_half(s, h):
            for d in (2, 1, 3):
                rdma = pltpu.make_async_remote_copy(
                    src_ref=ag_stage_ref.at[s, rows(h), :],
                    dst_ref=rbuf_ref.at[0, N_DEV - d, rows(h), :],
                    send_sem=ag_ssems.at[s, d - 1, h],
                    recv_sem=recv_sems.at[0, N_DEV - d, h],
                    device_id=(lax.rem(my + d, N_DEV),),
                    device_id_type=pl.DeviceIdType.MESH,
                )
                rdma.start()

        def ag_send_wait(s):
            for d in (1, 2, 3):
                for h in (0, 1):
                    ref = ag_stage_ref.at[s, rows(h), :]
                    pltpu.make_async_remote_copy(
                        src_ref=ref, dst_ref=ref,
                        send_sem=ag_ssems.at[s, d - 1, h],
                        recv_sem=recv_sems.at[0, 0, 0],
                        device_id=(my,),
                        device_id_type=pl.DeviceIdType.MESH,
                    ).wait_send()

        def rs_send_wait():
            for r in (1, 2, 3):
                for h in (0, 1):
                    ref = rs_stage_ref.at[r - 1, rows(h), :]
                    pltpu.make_async_remote_copy(
                        src_ref=ref, dst_ref=ref,
                        send_sem=rs_ssems.at[r - 1, h],
                        recv_sem=recv_sems.at[0, 0, 0],
                        device_id=(my,),
                        device_id_type=pl.DeviceIdType.MESH,
                    ).wait_send()

        def recv_wait(par, slot, h):
            ref = rbuf_ref.at[par, slot, rows(h), :]
            pltpu.make_async_remote_copy(
                src_ref=ref, dst_ref=ref,
                send_sem=ag_ssems.at[0, 0, 0],
                recv_sem=recv_sems.at[par, slot, h],
                device_id=(my,),
                device_id_type=pl.DeviceIdType.MESH,
            ).wait_recv()

        for h in (0, 1):
            ag_stage_ref[0, rows(h), :] = x_ref[rows(h), :].astype(jnp.bfloat16)
            ag_send_half(0, h)

        layers = ((win0_ref, wout0_ref), (win1_ref, wout1_ref),
                  (win2_ref, wout2_ref))
        for l, (win_ref, wout_ref) in enumerate(layers):
            s = l % 2
            s2 = (l + 1) % 2

            winb_ref[...] = win_ref[...].astype(jnp.bfloat16)
            woutb_ref[...] = wout_ref[...].astype(jnp.bfloat16)

            if l >= 1:
                rs_send_wait()

            def block_rows(xv_bf16, r, h):
                hid = jnp.maximum(
                    jnp.dot(xv_bf16, winb_ref[...],
                            preferred_element_type=jnp.float32),
                    0.0,
                )
                part_ref[r, rows(h), :] = jnp.dot(
                    hid.astype(jnp.bfloat16), woutb_ref[...],
                    preferred_element_type=jnp.float32)

            for h in (0, 1):
                block_rows(ag_stage_ref[s, rows(h), :], 0, h)

            for r, h in ((1, 0), (3, 0), (1, 1), (3, 1), (2, 0), (2, 1)):
                recv_wait(0, r, h)
                block_rows(rbuf_ref[0, r, rows(h), :], r, h)
                rs_stage_ref[r - 1, rows(h), :] = (
                    part_ref[r, rows(h), :].astype(jnp.bfloat16))
                rdma = pltpu.make_async_remote_copy(
                    src_ref=rs_stage_ref.at[r - 1, rows(h), :],
                    dst_ref=rbuf_ref.at[1, N_DEV - r, rows(h), :],
                    send_sem=rs_ssems.at[r - 1, h],
                    recv_sem=recv_sems.at[1, N_DEV - r, h],
                    device_id=(lax.rem(my + r, N_DEV),),
                    device_id_type=pl.DeviceIdType.MESH,
                )
                rdma.start()

            if l >= 1:
                ag_send_wait(s2)
            for h in (0, 1):
                for r in (1, 2, 3):
                    recv_wait(1, r, h)
                reduced = (
                    part_ref[0, rows(h), :]
                    + rbuf_ref[1, 1, rows(h), :].astype(jnp.float32)
                    + rbuf_ref[1, 2, rows(h), :].astype(jnp.float32)
                    + rbuf_ref[1, 3, rows(h), :].astype(jnp.float32)
                )
                if l < 2:
                    ag_stage_ref[s2, rows(h), :] = reduced.astype(jnp.bfloat16)
                    ag_send_half(s2, h)
                else:
                    out_ref[rows(h), :] = reduced

        rs_send_wait()
        ag_send_wait(0)

    return pl.pallas_call(
        body,
        out_shape=jax.ShapeDtypeStruct((M, D), jnp.float32),
        in_specs=[pl.BlockSpec(memory_space=pltpu.VMEM)] * 7,
        out_specs=pl.BlockSpec(memory_space=pltpu.VMEM),
        scratch_shapes=[
            pltpu.VMEM((N_DEV, M, D), jnp.float32),
            pltpu.VMEM((2, N_DEV, M, D), jnp.bfloat16),
            pltpu.VMEM((2, M, D), jnp.bfloat16),
            pltpu.VMEM((3, M, D), jnp.bfloat16),
            pltpu.VMEM((D, 2 * D), jnp.bfloat16),
            pltpu.VMEM((2 * D, D), jnp.bfloat16),
            pltpu.SemaphoreType.DMA((2, 3, 2)),
            pltpu.SemaphoreType.DMA((3, 2)),
            pltpu.SemaphoreType.DMA((2, N_DEV, 2)),
        ],
        compiler_params=pltpu.CompilerParams(collective_id=0),
    )(x, Win0, Wout0, Win1, Wout1, Win2, Wout2)


# device time: 35602 ns/iter; 1.1875x vs baseline; 1.0796x over previous
import jax
import jax.numpy as jnp
from jax import lax
from jax.experimental import pallas as pl
from jax.experimental.pallas import tpu as pltpu

N_DEV = 4
M = 256
Q = 64
D = 256


def kernel(x, Win0, Wout0, Win1, Wout1, Win2, Wout2):
    def body(x_ref, win0_ref, wout0_ref, win1_ref, wout1_ref, win2_ref,
             wout2_ref, out_ref, part_ref, rbuf_ref, ag_stage_ref,
             rs_stage_ref, winb_ref, woutb_ref,
             ag_ssems, rs_ssems, recv_sems):
        my = lax.axis_index("i")

        def rows(q):
            return pl.ds(q * Q, Q)

        barrier_sem = pltpu.get_barrier_semaphore()
        for d in (1, 2, 3):
            pl.semaphore_signal(
                barrier_sem, inc=1,
                device_id=(lax.rem(my + d, N_DEV),),
                device_id_type=pl.DeviceIdType.MESH,
            )
        pl.semaphore_wait(barrier_sem, N_DEV - 1)

        def ag_send_q(s, q):
            for d in (2, 1, 3):
                rdma = pltpu.make_async_remote_copy(
                    src_ref=ag_stage_ref.at[s, rows(q), :],
                    dst_ref=rbuf_ref.at[0, N_DEV - d, rows(q), :],
                    send_sem=ag_ssems.at[s, d - 1, q],
                    recv_sem=recv_sems.at[0, N_DEV - d, q],
                    device_id=(lax.rem(my + d, N_DEV),),
                    device_id_type=pl.DeviceIdType.MESH,
                )
                rdma.start()

        def ag_send_wait(s):
            for d in (1, 2, 3):
                for q in (0, 1, 2, 3):
                    ref = ag_stage_ref.at[s, rows(q), :]
                    pltpu.make_async_remote_copy(
                        src_ref=ref, dst_ref=ref,
                        send_sem=ag_ssems.at[s, d - 1, q],
                        recv_sem=recv_sems.at[0, 0, 0],
                        device_id=(my,),
                        device_id_type=pl.DeviceIdType.MESH,
                    ).wait_send()

        def rs_send_wait():
            for r in (1, 2, 3):
                for q in (0, 1, 2, 3):
                    ref = rs_stage_ref.at[r - 1, rows(q), :]
                    pltpu.make_async_remote_copy(
                        src_ref=ref, dst_ref=ref,
                        send_sem=rs_ssems.at[r - 1, q],
                        recv_sem=recv_sems.at[0, 0, 0],
                        device_id=(my,),
                        device_id_type=pl.DeviceIdType.MESH,
                    ).wait_send()

        def recv_wait(par, slot, q):
            ref = rbuf_ref.at[par, slot, rows(q), :]
            pltpu.make_async_remote_copy(
                src_ref=ref, dst_ref=ref,
                send_sem=ag_ssems.at[0, 0, 0],
                recv_sem=recv_sems.at[par, slot, q],
                device_id=(my,),
                device_id_type=pl.DeviceIdType.MESH,
            ).wait_recv()

        for q in (0, 1, 2, 3):
            ag_stage_ref[0, rows(q), :] = x_ref[rows(q), :].astype(jnp.bfloat16)
            ag_send_q(0, q)

        layers = ((win0_ref, wout0_ref), (win1_ref, wout1_ref),
                  (win2_ref, wout2_ref))
        for l, (win_ref, wout_ref) in enumerate(layers):
            s = l % 2
            s2 = (l + 1) % 2

            winb_ref[...] = win_ref[...].astype(jnp.bfloat16)
            woutb_ref[...] = wout_ref[...].astype(jnp.bfloat16)

            if l >= 1:
                rs_send_wait()

            def block_rows(xv_bf16, r, q):
                hid = jnp.maximum(
                    jnp.dot(xv_bf16, winb_ref[...],
                            preferred_element_type=jnp.float32),
                    0.0,
                )
                part_ref[r, rows(q), :] = jnp.dot(
                    hid.astype(jnp.bfloat16), woutb_ref[...],
                    preferred_element_type=jnp.float32)

            for q in (0, 1, 2, 3):
                block_rows(ag_stage_ref[s, rows(q), :], 0, q)

            for r, q in ((1, 0), (3, 0), (1, 1), (3, 1), (2, 0),
                         (1, 2), (3, 2), (2, 1), (1, 3), (3, 3),
                         (2, 2), (2, 3)):
                recv_wait(0, r, q)
                block_rows(rbuf_ref[0, r, rows(q), :], r, q)
                rs_stage_ref[r - 1, rows(q), :] = (
                    part_ref[r, rows(q), :].astype(jnp.bfloat16))
                rdma = pltpu.make_async_remote_copy(
                    src_ref=rs_stage_ref.at[r - 1, rows(q), :],
                    dst_ref=rbuf_ref.at[1, N_DEV - r, rows(q), :],
                    send_sem=rs_ssems.at[r - 1, q],
                    recv_sem=recv_sems.at[1, N_DEV - r, q],
                    device_id=(lax.rem(my + r, N_DEV),),
                    device_id_type=pl.DeviceIdType.MESH,
                )
                rdma.start()

            if l >= 1:
                ag_send_wait(s2)
            for q in (0, 1, 2, 3):
                recv_wait(1, 1, q)
                recv_wait(1, 3, q)
                psum = (
                    part_ref[0, rows(q), :]
                    + rbuf_ref[1, 1, rows(q), :].astype(jnp.float32)
                    + rbuf_ref[1, 3, rows(q), :].astype(jnp.float32)
                )
                recv_wait(1, 2, q)
                reduced = psum + rbuf_ref[1, 2, rows(q), :].astype(jnp.float32)
                if l < 2:
                    ag_stage_ref[s2, rows(q), :] = reduced.astype(jnp.bfloat16)
                    ag_send_q(s2, q)
                else:
                    out_ref[rows(q), :] = reduced

        rs_send_wait()
        ag_send_wait(0)

    return pl.pallas_call(
        body,
        out_shape=jax.ShapeDtypeStruct((M, D), jnp.float32),
        in_specs=[pl.BlockSpec(memory_space=pltpu.VMEM)] * 7,
        out_specs=pl.BlockSpec(memory_space=pltpu.VMEM),
        scratch_shapes=[
            pltpu.VMEM((N_DEV, M, D), jnp.float32),
            pltpu.VMEM((2, N_DEV, M, D), jnp.bfloat16),
            pltpu.VMEM((2, M, D), jnp.bfloat16),
            pltpu.VMEM((3, M, D), jnp.bfloat16),
            pltpu.VMEM((D, 2 * D), jnp.bfloat16),
            pltpu.VMEM((2 * D, D), jnp.bfloat16),
            pltpu.SemaphoreType.DMA((2, 3, 4)),
            pltpu.SemaphoreType.DMA((3, 4)),
            pltpu.SemaphoreType.DMA((2, N_DEV, 4)),
        ],
        compiler_params=pltpu.CompilerParams(collective_id=0),
    )(x, Win0, Wout0, Win1, Wout1, Win2, Wout2)


# device time: 35573 ns/iter; 1.1885x vs baseline; 1.0008x over previous
import jax
import jax.numpy as jnp
from jax import lax
from jax.experimental import pallas as pl
from jax.experimental.pallas import tpu as pltpu

N_DEV = 4
M = 256
Q = 64
D = 256


def kernel(x, Win0, Wout0, Win1, Wout1, Win2, Wout2):
    def body(x_ref, win0_ref, wout0_ref, win1_ref, wout1_ref, win2_ref,
             wout2_ref, out_ref, part0_ref, rbuf_ref, ag_stage_ref,
             rs_stage_ref, winb_ref, woutb_ref,
             ag_ssems, rs_ssems, recv_sems):
        my = lax.axis_index("i")

        def rows(q):
            return pl.ds(q * Q, Q)

        barrier_sem = pltpu.get_barrier_semaphore()
        for d in (1, 2, 3):
            pl.semaphore_signal(
                barrier_sem, inc=1,
                device_id=(lax.rem(my + d, N_DEV),),
                device_id_type=pl.DeviceIdType.MESH,
            )
        pl.semaphore_wait(barrier_sem, N_DEV - 1)

        def ag_send_q(s, q):
            for d in (2, 1, 3):
                rdma = pltpu.make_async_remote_copy(
                    src_ref=ag_stage_ref.at[s, rows(q), :],
                    dst_ref=rbuf_ref.at[0, N_DEV - d, rows(q), :],
                    send_sem=ag_ssems.at[s, d - 1, q],
                    recv_sem=recv_sems.at[0, N_DEV - d, q],
                    device_id=(lax.rem(my + d, N_DEV),),
                    device_id_type=pl.DeviceIdType.MESH,
                )
                rdma.start()

        def ag_send_wait(s):
            for d in (1, 2, 3):
                for q in (0, 1, 2, 3):
                    ref = ag_stage_ref.at[s, rows(q), :]
                    pltpu.make_async_remote_copy(
                        src_ref=ref, dst_ref=ref,
                        send_sem=ag_ssems.at[s, d - 1, q],
                        recv_sem=recv_sems.at[0, 0, 0],
                        device_id=(my,),
                        device_id_type=pl.DeviceIdType.MESH,
                    ).wait_send()

        def rs_send_wait():
            for r in (1, 2, 3):
                for q in (0, 1, 2, 3):
                    ref = rs_stage_ref.at[r - 1, rows(q), :]
                    pltpu.make_async_remote_copy(
                        src_ref=ref, dst_ref=ref,
                        send_sem=rs_ssems.at[r - 1, q],
                        recv_sem=recv_sems.at[0, 0, 0],
                        device_id=(my,),
                        device_id_type=pl.DeviceIdType.MESH,
                    ).wait_send()

        def recv_wait(par, slot, q):
            ref = rbuf_ref.at[par, slot, rows(q), :]
            pltpu.make_async_remote_copy(
                src_ref=ref, dst_ref=ref,
                send_sem=ag_ssems.at[0, 0, 0],
                recv_sem=recv_sems.at[par, slot, q],
                device_id=(my,),
                device_id_type=pl.DeviceIdType.MESH,
            ).wait_recv()

        for q in (0, 1, 2, 3):
            ag_stage_ref[0, rows(q), :] = x_ref[rows(q), :].astype(jnp.bfloat16)
            ag_send_q(0, q)

        layers = ((win0_ref, wout0_ref), (win1_ref, wout1_ref),
                  (win2_ref, wout2_ref))
        for l, (win_ref, wout_ref) in enumerate(layers):
            winb_ref[l, :, :] = win_ref[...].astype(jnp.bfloat16)
            woutb_ref[l, :, :] = wout_ref[...].astype(jnp.bfloat16)

        for l in range(3):
            s = l % 2
            s2 = (l + 1) % 2

            if l >= 1:
                rs_send_wait()

            def block_rows(xv_bf16, r, q, l=l):
                hid = jnp.maximum(
                    jnp.dot(xv_bf16, winb_ref[l, :, :],
                            preferred_element_type=jnp.float32),
                    0.0,
                )
                pv = jnp.dot(hid.astype(jnp.bfloat16), woutb_ref[l, :, :],
                             preferred_element_type=jnp.float32)
                if r == 0:
                    part0_ref[rows(q), :] = pv
                else:
                    rs_stage_ref[r - 1, rows(q), :] = pv.astype(jnp.bfloat16)

            for q in (0, 1, 2, 3):
                block_rows(ag_stage_ref[s, rows(q), :], 0, q)

            for r, q in ((1, 0), (3, 0), (1, 1), (3, 1), (2, 0),
                         (1, 2), (3, 2), (2, 1), (1, 3), (3, 3),
                         (2, 2), (2, 3)):
                recv_wait(0, r, q)
                block_rows(rbuf_ref[0, r, rows(q), :], r, q)
                rdma = pltpu.make_async_remote_copy(
                    src_ref=rs_stage_ref.at[r - 1, rows(q), :],
                    dst_ref=rbuf_ref.at[1, N_DEV - r, rows(q), :],
                    send_sem=rs_ssems.at[r - 1, q],
                    recv_sem=recv_sems.at[1, N_DEV - r, q],
                    device_id=(lax.rem(my + r, N_DEV),),
                    device_id_type=pl.DeviceIdType.MESH,
                )
                rdma.start()

            if l >= 1:
                ag_send_wait(s2)
            for q in (0, 1, 2, 3):
                recv_wait(1, 1, q)
                recv_wait(1, 3, q)
                psum = (
                    part0_ref[rows(q), :]
                    + rbuf_ref[1, 1, rows(q), :].astype(jnp.float32)
                    + rbuf_ref[1, 3, rows(q), :].astype(jnp.float32)
                )
                recv_wait(1, 2, q)
                reduced = psum + rbuf_ref[1, 2, rows(q), :].astype(jnp.float32)
                if l < 2:
                    ag_stage_ref[s2, rows(q), :] = reduced.astype(jnp.bfloat16)
                    ag_send_q(s2, q)
                else:
                    out_ref[rows(q), :] = reduced

        rs_send_wait()
        ag_send_wait(0)

    return pl.pallas_call(
        body,
        out_shape=jax.ShapeDtypeStruct((M, D), jnp.float32),
        in_specs=[pl.BlockSpec(memory_space=pltpu.VMEM)] * 7,
        out_specs=pl.BlockSpec(memory_space=pltpu.VMEM),
        scratch_shapes=[
            pltpu.VMEM((M, D), jnp.float32),
            pltpu.VMEM((2, N_DEV, M, D), jnp.bfloat16),
            pltpu.VMEM((2, M, D), jnp.bfloat16),
            pltpu.VMEM((3, M, D), jnp.bfloat16),
            pltpu.VMEM((3, D, 2 * D), jnp.bfloat16),
            pltpu.VMEM((3, 2 * D, D), jnp.bfloat16),
            pltpu.SemaphoreType.DMA((2, 3, 4)),
            pltpu.SemaphoreType.DMA((3, 4)),
            pltpu.SemaphoreType.DMA((2, N_DEV, 4)),
        ],
        compiler_params=pltpu.CompilerParams(collective_id=0),
    )(x, Win0, Wout0, Win1, Wout1, Win2, Wout2)
